# baseline (speedup 1.0000x reference)
import functools

import jax
import jax.numpy as jnp
from jax import lax
from jax.experimental import pallas as pl
from jax.experimental.pallas import tpu as pltpu
from jax.experimental.pallas import tpu_sc as plsc


def kernel(x, seq_emb_weight, pos_encoding):
    B, S = x.shape
    V, D = seq_emb_weight.shape
    N = B * S

    info = plsc.get_sparse_core_info()
    NC, NS, L = info.num_cores, info.num_subcores, info.num_lanes
    NW = NC * NS

    CHUNK = 128
    per_w = N // NW
    n_chunks = per_w // CHUNK

    pos = pos_encoding[:S]

    mesh = plsc.VectorSubcoreMesh(core_axis_name="c", subcore_axis_name="s",
                                  num_cores=NC)

    @functools.partial(
        pl.kernel,
        mesh=mesh,
        out_type=jax.ShapeDtypeStruct((N, D), jnp.float32),
        compiler_params=pltpu.CompilerParams(use_tc_tiling_on_sc=False),
        scratch_types=[
            pltpu.VMEM((CHUNK,), jnp.int32),
            pltpu.VMEM((CHUNK, D), jnp.float32),
            pltpu.VMEM((2 * S, D), jnp.float32),
            pltpu.SemaphoreType.DMA,
        ],
    )
    def emb_pos_kernel(table_hbm, idx_hbm, pos_hbm, out_hbm,
                       idx_v, rows_v, pos2_v, sem):
        wid = lax.axis_index("s") * NC + lax.axis_index("c")
        base0 = wid * per_w

        pltpu.sync_copy(pos_hbm, pos2_v.at[pl.ds(0, S)])
        pltpu.sync_copy(pos_hbm, pos2_v.at[pl.ds(S, S)])

        def chunk_body(c, carry):
            base = base0 + c * CHUNK
            pltpu.sync_copy(idx_hbm.at[pl.ds(base, CHUNK)], idx_v)
            pltpu.async_copy(table_hbm.at[idx_v], rows_v, sem).wait()
            s0 = lax.rem(c * CHUNK, S)

            def row_body(r, rcarry):
                pr = s0 + r
                for j in range(D // L):
                    pv = pos2_v[pr, pl.ds(j * L, L)]
                    plsc.addupdate(rows_v.at[r, pl.ds(j * L, L)], pv)
                return rcarry

            lax.fori_loop(0, CHUNK, row_body, 0)
            pltpu.sync_copy(rows_v, out_hbm.at[pl.ds(base, CHUNK)])
            return carry

        lax.fori_loop(0, n_chunks, chunk_body, 0)

    out = emb_pos_kernel(seq_emb_weight, x.reshape(N), pos)
    return out.reshape(B, S, D)


# speedup vs baseline: 1.2335x; 1.2335x over previous
import functools

import jax
import jax.numpy as jnp
from jax import lax
from jax.experimental import pallas as pl
from jax.experimental.pallas import tpu as pltpu
from jax.experimental.pallas import tpu_sc as plsc


def kernel(x, seq_emb_weight, pos_encoding):
    B, S = x.shape
    V, D = seq_emb_weight.shape
    N = B * S

    info = plsc.get_sparse_core_info()
    NC, NS, L = info.num_cores, info.num_subcores, info.num_lanes
    NW = NC * NS

    CHUNK = 128
    NBUF = 4
    GD = 2
    per_w = N // NW
    n_chunks = per_w // CHUNK
    n_groups = n_chunks // NBUF

    pos = pos_encoding[:S]

    mesh = plsc.VectorSubcoreMesh(core_axis_name="c", subcore_axis_name="s",
                                  num_cores=NC)

    @functools.partial(
        pl.kernel,
        mesh=mesh,
        out_type=jax.ShapeDtypeStruct((N, D), jnp.float32),
        compiler_params=pltpu.CompilerParams(use_tc_tiling_on_sc=False),
        scratch_types=[
            pltpu.VMEM((NBUF, CHUNK), jnp.int32),
            pltpu.VMEM((NBUF, CHUNK, D), jnp.float32),
            pltpu.VMEM((2 * S, D), jnp.float32),
            pltpu.SemaphoreType.DMA((NBUF,)),
            pltpu.SemaphoreType.DMA((NBUF,)),
            pltpu.SemaphoreType.DMA((NBUF,)),
        ],
    )
    def emb_pos_kernel(table_hbm, idx_hbm, pos_hbm, out_hbm,
                       idx_v, rows_v, pos2_v, sem_idx, sem_g, sem_out):
        wid = lax.axis_index("s") * NC + lax.axis_index("c")
        base0 = wid * per_w

        pltpu.sync_copy(pos_hbm, pos2_v.at[pl.ds(0, S)])
        pltpu.sync_copy(pos_hbm, pos2_v.at[pl.ds(S, S)])

        def idx_copy(c, b):
            return pltpu.make_async_copy(
                idx_hbm.at[pl.ds(base0 + c * CHUNK, CHUNK)],
                idx_v.at[b], sem_idx.at[b])

        def gather_copy(b):
            return pltpu.make_async_copy(
                table_hbm.at[idx_v.at[b]], rows_v.at[b], sem_g.at[b])

        def out_copy(c, b):
            return pltpu.make_async_copy(
                rows_v.at[b],
                out_hbm.at[pl.ds(base0 + c * CHUNK, CHUNK)], sem_out.at[b])

        for b in range(NBUF):
            idx_copy(b, b).start()
        for c in range(GD):
            idx_copy(c, c).wait()
            gather_copy(c).start()

        def group_body(g, carry):
            for b in range(NBUF):
                c = g * NBUF + b
                gather_copy(b).wait()

                @pl.when(c + NBUF < n_chunks)
                def _():
                    idx_copy(c + NBUF, b).start()

                s0 = lax.rem(c * CHUNK, S)

                def row_body(r, rcarry):
                    pr = s0 + r
                    for j in range(D // L):
                        plsc.addupdate(rows_v.at[b, r, pl.ds(j * L, L)],
                                       pos2_v[pr, pl.ds(j * L, L)])
                    return rcarry

                lax.fori_loop(0, CHUNK, row_body, 0)

                out_copy(c, b).start()

                b2 = (b + GD) % NBUF

                @pl.when(c + GD < n_chunks)
                def _():
                    idx_copy(c + GD, b2).wait()

                    @pl.when(c + GD >= NBUF)
                    def _():
                        out_copy(0, b2).wait()

                    gather_copy(b2).start()
            return carry

        lax.fori_loop(0, n_groups, group_body, 0)

        for b in range(NBUF):
            out_copy(0, b).wait()

    out = emb_pos_kernel(seq_emb_weight, x.reshape(N), pos)
    return out.reshape(B, S, D)


# speedup vs baseline: 1.5068x; 1.2216x over previous
import functools

import jax
import jax.numpy as jnp
from jax import lax
from jax.experimental import pallas as pl
from jax.experimental.pallas import tpu as pltpu
from jax.experimental.pallas import tpu_sc as plsc


def kernel(x, seq_emb_weight, pos_encoding):
    B, S = x.shape
    V, D = seq_emb_weight.shape
    N = B * S

    info = plsc.get_sparse_core_info()
    NC, NS, L = info.num_cores, info.num_subcores, info.num_lanes
    NW = NC * NS

    NBUF = 4
    GD = 2
    seq_per_w = B // NW
    n_groups = seq_per_w // NBUF
    G1 = 128
    G2 = S - G1

    pos = pos_encoding[:S]

    mesh = plsc.VectorSubcoreMesh(core_axis_name="c", subcore_axis_name="s",
                                  num_cores=NC)

    @functools.partial(
        pl.kernel,
        mesh=mesh,
        out_type=jax.ShapeDtypeStruct((B, S, D), jnp.float32),
        compiler_params=pltpu.CompilerParams(use_tc_tiling_on_sc=False),
        scratch_types=[
            pltpu.VMEM((NBUF, 2, G1), jnp.int32),
            pltpu.VMEM((NBUF, S, D), jnp.float32),
            pltpu.VMEM((S, D), jnp.float32),
            pltpu.SemaphoreType.DMA((NBUF,)),
            pltpu.SemaphoreType.DMA((NBUF,)),
            pltpu.SemaphoreType.DMA((NBUF,)),
        ],
    )
    def emb_pos_kernel(table_hbm, idx_hbm, pos_hbm, out_hbm,
                       idx_v, rows_v, pos_v, sem_idx, sem_g, sem_out):
        wid = lax.axis_index("s") * NC + lax.axis_index("c")
        seq0 = wid * seq_per_w

        pltpu.sync_copy(pos_hbm, pos_v)

        def idx_copies(s, b):
            fb = (seq0 + s) * S
            return (
                pltpu.make_async_copy(
                    idx_hbm.at[pl.ds(fb, G1)], idx_v.at[b, 0], sem_idx.at[b]),
                pltpu.make_async_copy(
                    idx_hbm.at[pl.ds(fb + G1, G2)],
                    idx_v.at[b, 1, pl.ds(0, G2)], sem_idx.at[b]),
            )

        def gather_copies(b):
            return (
                pltpu.make_async_copy(
                    table_hbm.at[idx_v.at[b, 0]],
                    rows_v.at[b, pl.ds(0, G1)], sem_g.at[b]),
                pltpu.make_async_copy(
                    table_hbm.at[idx_v.at[b, 1, pl.ds(0, G2)]],
                    rows_v.at[b, pl.ds(G1, G2)], sem_g.at[b]),
            )

        def out_copy(s, b):
            return pltpu.make_async_copy(
                rows_v.at[b], out_hbm.at[seq0 + s], sem_out.at[b])

        def start2(copies):
            copies[0].start()
            copies[1].start()

        def wait2(copies):
            copies[0].wait()
            copies[1].wait()

        for b in range(NBUF):
            start2(idx_copies(b, b))
        for s in range(GD):
            wait2(idx_copies(s, s))
            start2(gather_copies(s))

        def group_body(g, carry):
            for b in range(NBUF):
                s = g * NBUF + b
                wait2(gather_copies(b))

                @pl.when(s + NBUF < seq_per_w)
                def _():
                    start2(idx_copies(s + NBUF, b))

                def row_body(r, rcarry):
                    for j in range(D // L):
                        plsc.addupdate(rows_v.at[b, r, pl.ds(j * L, L)],
                                       pos_v[r, pl.ds(j * L, L)])
                    return rcarry

                lax.fori_loop(0, S, row_body, 0)

                out_copy(s, b).start()

                b2 = (b + GD) % NBUF

                @pl.when(s + GD < seq_per_w)
                def _():
                    wait2(idx_copies(s + GD, b2))

                    @pl.when(s + GD >= NBUF)
                    def _():
                        out_copy(0, b2).wait()

                    start2(gather_copies(b2))
            return carry

        lax.fori_loop(0, n_groups, group_body, 0)

        for b in range(NBUF):
            out_copy(0, b).wait()

    return emb_pos_kernel(seq_emb_weight, x.reshape(N), pos)


# speedup vs baseline: 1.5256x; 1.0124x over previous
import functools

import jax
import jax.numpy as jnp
from jax import lax
from jax.experimental import pallas as pl
from jax.experimental.pallas import tpu as pltpu
from jax.experimental.pallas import tpu_sc as plsc


def kernel(x, seq_emb_weight, pos_encoding):
    B, S = x.shape
    V, D = seq_emb_weight.shape
    N = B * S

    info = plsc.get_sparse_core_info()
    NC, NS, L = info.num_cores, info.num_subcores, info.num_lanes
    NW = NC * NS

    NBUF = 4
    GD = 2
    seq_per_w = B // NW
    n_groups = seq_per_w // NBUF
    G1 = 128
    G2 = S - G1

    pos = pos_encoding[:S]

    mesh = plsc.VectorSubcoreMesh(core_axis_name="c", subcore_axis_name="s",
                                  num_cores=NC)

    @functools.partial(
        pl.kernel,
        mesh=mesh,
        out_type=jax.ShapeDtypeStruct((B, S, D), jnp.float32),
        compiler_params=pltpu.CompilerParams(use_tc_tiling_on_sc=False),
        scratch_types=[
            pltpu.VMEM((NBUF, 2, G1), jnp.int32),
            pltpu.VMEM((NBUF, S, D), jnp.float32),
            pltpu.VMEM((S, D), jnp.float32),
            pltpu.SemaphoreType.DMA((NBUF,)),
            pltpu.SemaphoreType.DMA((NBUF,)),
            pltpu.SemaphoreType.DMA((NBUF,)),
        ],
    )
    def emb_pos_kernel(table_hbm, idx_hbm, pos_hbm, out_hbm,
                       idx_v, rows_v, pos_v, sem_idx, sem_g, sem_out):
        wid = lax.axis_index("s") * NC + lax.axis_index("c")
        seq0 = wid * seq_per_w

        pltpu.sync_copy(pos_hbm, pos_v)

        def idx_copies(s, b):
            fb = (seq0 + s) * S
            return (
                pltpu.make_async_copy(
                    idx_hbm.at[pl.ds(fb, G1)], idx_v.at[b, 0], sem_idx.at[b]),
                pltpu.make_async_copy(
                    idx_hbm.at[pl.ds(fb + G1, G2)],
                    idx_v.at[b, 1, pl.ds(0, G2)], sem_idx.at[b]),
            )

        def gather_copies(b):
            return (
                pltpu.make_async_copy(
                    table_hbm.at[idx_v.at[b, 0]],
                    rows_v.at[b, pl.ds(0, G1)], sem_g.at[b]),
                pltpu.make_async_copy(
                    table_hbm.at[idx_v.at[b, 1, pl.ds(0, G2)]],
                    rows_v.at[b, pl.ds(G1, G2)], sem_g.at[b]),
            )

        def out_copy(s, b):
            return pltpu.make_async_copy(
                rows_v.at[b], out_hbm.at[seq0 + s], sem_out.at[b])

        def start2(copies):
            copies[0].start()
            copies[1].start()

        def wait2(copies):
            copies[0].wait()
            copies[1].wait()

        for b in range(NBUF):
            start2(idx_copies(b, b))
        for s in range(GD):
            wait2(idx_copies(s, s))
            start2(gather_copies(s))

        def group_body(g, carry):
            for b in range(NBUF):
                s = g * NBUF + b
                wait2(gather_copies(b))

                @pl.when(s + NBUF < seq_per_w)
                def _():
                    start2(idx_copies(s + NBUF, b))

                b2 = (b + GD) % NBUF

                @pl.when(s + GD < seq_per_w)
                def _():
                    wait2(idx_copies(s + GD, b2))

                    @pl.when(s + GD >= NBUF)
                    def _():
                        out_copy(0, b2).wait()

                    start2(gather_copies(b2))

                def row_body(i, rcarry):
                    for rr in range(4):
                        r = i * 4 + rr
                        for j in range(D // L):
                            plsc.addupdate(rows_v.at[b, r, pl.ds(j * L, L)],
                                           pos_v[r, pl.ds(j * L, L)])
                    return rcarry

                lax.fori_loop(0, S // 4, row_body, 0)

                out_copy(s, b).start()
            return carry

        lax.fori_loop(0, n_groups, group_body, 0)

        for b in range(NBUF):
            out_copy(0, b).wait()

    return emb_pos_kernel(seq_emb_weight, x.reshape(N), pos)
